# baseline (speedup 1.0000x reference)
import functools

import jax
import jax.numpy as jnp
from jax import lax
from jax.experimental import pallas as pl
from jax.experimental.pallas import tpu as pltpu
from jax.experimental.pallas import tpu_sc as plsc

R = 128
N = 32768
NC = 2
NS = 16
L = 16
NW = NC * NS
CPW = N // NW
CHUNK = 512
NCHUNK = CPW // CHUNK


def _sc_argmax(x):
    mesh = plsc.VectorSubcoreMesh(core_axis_name="c", subcore_axis_name="s")

    @functools.partial(
        pl.kernel,
        out_type=jax.ShapeDtypeStruct((N,), jnp.int32),
        mesh=mesh,
        scratch_types=[
            pltpu.VMEM((R, CHUNK), jnp.float32),
            pltpu.VMEM((CPW,), jnp.int32),
        ],
    )
    def k(x_hbm, out_hbm, buf, idxbuf):
        wid = lax.axis_index("s") * NC + lax.axis_index("c")
        base = wid * CPW

        def chunk_body(cidx, _):
            col0 = base + cidx * CHUNK
            pltpu.sync_copy(x_hbm.at[:, pl.ds(col0, CHUNK)], buf)

            def group_body(g, _):
                off = g * L

                def row_body(r, carry):
                    vmax, vidx = carry
                    v = buf[r, pl.ds(off, L)]
                    pred = v > vmax
                    vmax = jnp.where(pred, v, vmax)
                    vidx = jnp.where(pred, jnp.full((L,), r, jnp.int32), vidx)
                    return vmax, vidx

                init = (buf[0, pl.ds(off, L)], jnp.zeros((L,), jnp.int32))
                _, vidx = lax.fori_loop(1, R, row_body, init)
                idxbuf[pl.ds(cidx * CHUNK + off, L)] = vidx
                return 0

            lax.fori_loop(0, CHUNK // L, group_body, 0)
            return 0

        lax.fori_loop(0, NCHUNK, chunk_body, 0)
        pltpu.sync_copy(idxbuf, out_hbm.at[pl.ds(base, CPW)])

    return k(x)


def kernel(x):
    return _sc_argmax(x).astype(jnp.int64)


# speedup vs baseline: 1.7761x; 1.7761x over previous
import functools

import jax
import jax.numpy as jnp
from jax import lax
from jax.experimental import pallas as pl
from jax.experimental.pallas import tpu as pltpu
from jax.experimental.pallas import tpu_sc as plsc

R = 128
N = 32768
NC = 2
NS = 16
L = 16
NW = NC * NS
CPW = N // NW
CHUNK = 256
NCHUNK = CPW // CHUNK
G = 4
NGB = CHUNK // (G * L)


def _sc_argmax(x):
    mesh = plsc.VectorSubcoreMesh(core_axis_name="c", subcore_axis_name="s")

    @functools.partial(
        pl.kernel,
        out_type=jax.ShapeDtypeStruct((N,), jnp.int32),
        mesh=mesh,
        scratch_types=[
            pltpu.VMEM((R, CHUNK), jnp.float32),
            pltpu.VMEM((R, CHUNK), jnp.float32),
            pltpu.VMEM((CPW,), jnp.int32),
            pltpu.SemaphoreType.DMA,
            pltpu.SemaphoreType.DMA,
        ],
    )
    def k(x_hbm, out_hbm, buf0, buf1, idxbuf, sem0, sem1):
        wid = lax.axis_index("s") * NC + lax.axis_index("c")
        base = wid * CPW
        bufs = (buf0, buf1)
        sems = (sem0, sem1)

        descs = [None, None]
        descs[0] = pltpu.async_copy(
            x_hbm.at[:, pl.ds(base, CHUNK)], bufs[0], sems[0])
        for c in range(NCHUNK):
            cur = c % 2
            descs[cur].wait()
            if c + 1 < NCHUNK:
                descs[1 - cur] = pltpu.async_copy(
                    x_hbm.at[:, pl.ds(base + (c + 1) * CHUNK, CHUNK)],
                    bufs[1 - cur], sems[1 - cur])
            buf = bufs[cur]
            for gb in range(NGB):
                offs = [gb * G * L + g * L for g in range(G)]
                init_max = tuple(buf[0, pl.ds(o, L)] for o in offs)
                init_idx = tuple(jnp.zeros((L,), jnp.int32) for _ in offs)

                @plsc.parallel_loop(1, R, 1, unroll=4,
                                    carry=(init_max, init_idx))
                def row_body(r, carry, buf=buf, offs=offs):
                    vmax, vidx = carry
                    ridx = jnp.full((L,), r, jnp.int32)
                    new_max = []
                    new_idx = []
                    for g in range(G):
                        v = buf[r, pl.ds(offs[g], L)]
                        pred = v > vmax[g]
                        new_max.append(jnp.where(pred, v, vmax[g]))
                        new_idx.append(jnp.where(pred, ridx, vidx[g]))
                    return tuple(new_max), tuple(new_idx)

                _, final_idx = row_body
                for g in range(G):
                    idxbuf[pl.ds(c * CHUNK + offs[g], L)] = final_idx[g]

        pltpu.sync_copy(idxbuf, out_hbm.at[pl.ds(base, CPW)])

    return k(x)


def kernel(x):
    return _sc_argmax(x).astype(jnp.int64)


# speedup vs baseline: 1.8341x; 1.0326x over previous
import functools

import jax
import jax.numpy as jnp
from jax import lax
from jax.experimental import pallas as pl
from jax.experimental.pallas import tpu as pltpu
from jax.experimental.pallas import tpu_sc as plsc

R = 128
N = 32768
NC = 2
NS = 16
L = 16
NW = NC * NS
CPW = N // NW
CHUNK = 256
NCHUNK = CPW // CHUNK
G = 4
NGB = CHUNK // (G * L)
UNROLL = 8


def _sc_argmax(x):
    mesh = plsc.VectorSubcoreMesh(core_axis_name="c", subcore_axis_name="s")

    @functools.partial(
        pl.kernel,
        out_type=jax.ShapeDtypeStruct((N,), jnp.int32),
        mesh=mesh,
        scratch_types=[
            pltpu.VMEM((R, CHUNK), jnp.float32),
            pltpu.VMEM((R, CHUNK), jnp.float32),
            pltpu.VMEM((CPW,), jnp.int32),
            pltpu.SemaphoreType.DMA,
            pltpu.SemaphoreType.DMA,
        ],
    )
    def k(x_hbm, out_hbm, buf0, buf1, idxbuf, sem0, sem1):
        wid = lax.axis_index("s") * NC + lax.axis_index("c")
        base = wid * CPW

        def start(chunk_idx, buf, sem):
            pltpu.async_copy(
                x_hbm.at[:, pl.ds(base + chunk_idx * CHUNK, CHUNK)], buf, sem)

        def wait(buf, sem):
            pltpu.make_async_copy(
                x_hbm.at[:, pl.ds(base, CHUNK)], buf, sem).wait()

        def compute(buf, out_off):
            def gb_body(gb, _):
                off = gb * (G * L)
                init = (
                    tuple(jnp.full((L,), -jnp.inf, jnp.float32)
                          for _ in range(G)),
                    tuple(jnp.zeros((L,), jnp.int32) for _ in range(G)),
                )

                @plsc.parallel_loop(0, R, 1, unroll=UNROLL, carry=init)
                def row_body(r, carry):
                    vmax, vidx = carry
                    ridx = jnp.full((L,), r, jnp.int32)
                    new_max = []
                    new_idx = []
                    for g in range(G):
                        v = buf[r, pl.ds(off + g * L, L)]
                        pred = v > vmax[g]
                        new_max.append(jnp.where(pred, v, vmax[g]))
                        new_idx.append(jnp.where(pred, ridx, vidx[g]))
                    return tuple(new_max), tuple(new_idx)

                _, final_idx = row_body
                for g in range(G):
                    idxbuf[pl.ds(out_off + off + g * L, L)] = final_idx[g]
                return 0

            lax.fori_loop(0, NGB, gb_body, 0)

        start(0, buf0, sem0)

        def pair_body(c, _):
            k0 = 2 * c
            wait(buf0, sem0)
            start(k0 + 1, buf1, sem1)
            compute(buf0, k0 * CHUNK)
            wait(buf1, sem1)

            @pl.when(k0 + 2 < NCHUNK)
            def _():
                start(k0 + 2, buf0, sem0)

            compute(buf1, (k0 + 1) * CHUNK)
            return 0

        lax.fori_loop(0, NCHUNK // 2, pair_body, 0)
        pltpu.sync_copy(idxbuf, out_hbm.at[pl.ds(base, CPW)])

    return k(x)


def kernel(x):
    return _sc_argmax(x).astype(jnp.int64)


# speedup vs baseline: 7.2131x; 3.9328x over previous
import functools
import jax, jax.numpy as jnp
from jax import lax
from jax.experimental import pallas as pl
from jax.experimental.pallas import tpu as pltpu

R = 128
N = 32768
BN = 8192
SL = 8
NCH = R // SL


def _tc_argmax_body(x_ref, o_ref):
    vmax = x_ref[0:SL, :]
    vchunk = jnp.zeros((SL, BN), jnp.int32)
    for c in range(1, NCH):
        v = x_ref[SL * c:SL * (c + 1), :]
        p = v > vmax
        vmax = jnp.where(p, v, vmax)
        vchunk = jnp.where(p, jnp.int32(c), vchunk)
    m = jnp.max(vmax, axis=0)
    srow = lax.broadcasted_iota(jnp.int32, (SL, BN), 0)
    cand = jnp.where(vmax == m[None, :],
                     (vchunk << 3) | srow,
                     jnp.int32(R))
    o_ref[...] = jnp.min(cand, axis=0)


def kernel(x):
    out = pl.pallas_call(
        _tc_argmax_body,
        out_shape=jax.ShapeDtypeStruct((N,), jnp.int32),
        grid=(N // BN,),
        in_specs=[pl.BlockSpec((R, BN), lambda i: (0, i))],
        out_specs=pl.BlockSpec((BN,), lambda i: (i,)),
    )(x)
    return out.astype(jnp.int64)
